# Initial kernel scaffold; baseline (speedup 1.0000x reference)
import jax
import jax.numpy as jnp
from jax.experimental import pallas as pl


def kernel(in_feat, edge_index, W1, b1, W2, b2):
    raise NotImplementedError("write your pallas kernel here")


# speedup vs baseline: 8.5411x; 8.5411x over previous
import functools

import jax
import jax.numpy as jnp
from jax import lax
from jax.experimental import pallas as pl
from jax.experimental.pallas import tpu as pltpu
from jax.experimental.pallas import tpu_sc as plsc

N_NODES = 10000
N_PAD = 10240
NC, NS = 2, 16
NW = NC * NS
CHUNK = 128
CPT = 80
EPT = CHUNK * CPT
E_PAD = EPT * NW
ROWS_PT = N_PAD // NS
DEG_W = 8

_mesh = functools.partial(
    plsc.VectorSubcoreMesh,
    core_axis_name="c", subcore_axis_name="s",
    num_cores=NC, num_subcores=NS,
)


def _deg_body(src_hbm, dst_hbm, out_hbm, src_v, dst_v, hist_src, hist_dst):
    c = lax.axis_index("c")
    s = lax.axis_index("s")
    wid = s * NC + c
    pltpu.sync_copy(src_hbm.at[pl.ds(wid * CPT, CPT)], src_v)
    pltpu.sync_copy(dst_hbm.at[pl.ds(wid * CPT, CPT)], dst_v)

    zv = jnp.zeros((16,), jnp.float32)

    def zloop(i, carry):
        hist_src[pl.ds(i * 16, 16)] = zv
        hist_dst[pl.ds(i * 16, 16)] = zv
        return carry

    lax.fori_loop(0, N_PAD // 16, zloop, 0)

    ones_v = jnp.ones((16,), jnp.float32)

    def eloop(j, carry):
        def klf(k, carry2):
            plsc.addupdate_scatter(hist_src, [src_v[j, pl.ds(k * 16, 16)]],
                                   ones_v)
            plsc.addupdate_scatter(hist_dst, [dst_v[j, pl.ds(k * 16, 16)]],
                                   ones_v)
            return carry2

        return lax.fori_loop(0, CHUNK // 16, klf, carry)

    lax.fori_loop(0, CPT, eloop, 0)
    pltpu.sync_copy(hist_src, out_hbm.at[c, s, 0])
    pltpu.sync_copy(hist_dst, out_hbm.at[c, s, 1])


def _make_deg_kernel():
    return pl.kernel(
        _deg_body,
        out_type=jax.ShapeDtypeStruct((NC, NS, 2, N_PAD), jnp.float32),
        mesh=_mesh(),
        compiler_params=pltpu.CompilerParams(needs_layout_passes=False),
        scratch_types=[
            pltpu.VMEM((CPT, CHUNK), jnp.int32),
            pltpu.VMEM((CPT, CHUNK), jnp.int32),
            pltpu.VMEM((N_PAD,), jnp.float32),
            pltpu.VMEM((N_PAD,), jnp.float32),
        ],
    )


def _agg_body(y_hbm, src_hbm, dst_hbm, zeros_hbm, out_hbm,
              src_v, dst_v, gb0, zer_v, acc, sem0):
    c = lax.axis_index("c")
    s = lax.axis_index("s")
    wid = s * NC + c
    pltpu.sync_copy(src_hbm.at[pl.ds(wid * CPT, CPT)], src_v)
    pltpu.sync_copy(dst_hbm.at[pl.ds(wid * CPT, CPT)], dst_v)
    pltpu.sync_copy(zeros_hbm, zer_v)
    base = s * ROWS_PT

    def zloop(i, carry):
        pltpu.sync_copy(zer_v, acc.at[pl.ds(base + i * CHUNK, CHUNK)])
        return carry

    lax.fori_loop(0, ROWS_PT // CHUNK, zloop, 0)
    plsc.subcore_barrier()

    def eloop(j, carry):
        pltpu.async_copy(y_hbm.at[src_v.at[j]], gb0, sem0).wait()
        pltpu.sync_copy(gb0, acc.at[dst_v.at[j]], add=True)
        return carry

    lax.fori_loop(0, CPT, eloop, 0)
    plsc.subcore_barrier()
    pltpu.sync_copy(acc.at[pl.ds(base, ROWS_PT)],
                    out_hbm.at[c, pl.ds(base, ROWS_PT)])


def _make_agg_kernel(d):
    return pl.kernel(
        _agg_body,
        out_type=jax.ShapeDtypeStruct((NC, N_PAD, d), jnp.float32),
        mesh=_mesh(),
        compiler_params=pltpu.CompilerParams(use_tc_tiling_on_sc=False),
        scratch_types=[
            pltpu.VMEM((CPT, CHUNK), jnp.int32),
            pltpu.VMEM((CPT, CHUNK), jnp.int32),
            pltpu.VMEM((CHUNK, d), jnp.float32),
            pltpu.VMEM((CHUNK, d), jnp.float32),
            pltpu.VMEM_SHARED((N_PAD, d), jnp.float32),
            pltpu.SemaphoreType.DMA,
        ],
    )


def _tc_a_body(x_ref, w1_ref, deg_ref, y1a_ref, y1b_ref, norms_ref):
    deg = jnp.sum(deg_ref[...], axis=(0, 1))
    norms = lax.rsqrt(jnp.maximum(deg, 1.0))
    norms_ref[...] = norms
    xs = x_ref[...] * norms[0][:, None]
    y1 = jnp.dot(xs, w1_ref[...], preferred_element_type=jnp.float32)
    half = y1.shape[1] // 2
    y1a_ref[...] = y1[:, :half]
    y1b_ref[...] = y1[:, half:]


def _tc_b_body(agga_ref, aggb_ref, norms_ref, b1_ref, w2_ref, y2_ref):
    agg = jnp.concatenate(
        [agga_ref[0] + agga_ref[1], aggb_ref[0] + aggb_ref[1]], axis=1)
    norms = norms_ref[...]
    h = jnp.maximum(agg * norms[1][:, None] + b1_ref[...], 0.0)
    y2_ref[...] = jnp.dot(h * norms[0][:, None], w2_ref[...],
                          preferred_element_type=jnp.float32)


def _tc_c_body(agg_ref, norms_ref, b2_ref, out_ref):
    agg = agg_ref[0] + agg_ref[1]
    res = agg * norms_ref[1][:, None] + b2_ref[...]
    out_ref[...] = res[:N_NODES]


@jax.jit
def kernel(in_feat, edge_index, W1, b1, W2, b2):
    f32 = jnp.float32
    src = edge_index[0].astype(jnp.int32)
    dst = edge_index[1].astype(jnp.int32)
    n_extra = E_PAD - src.shape[0]
    pad_idx = N_NODES + (jnp.arange(n_extra, dtype=jnp.int32) % CHUNK)
    src_p = jnp.concatenate([src, pad_idx]).reshape(NW * CPT, CHUNK)
    dst_p = jnp.concatenate([dst, pad_idx]).reshape(NW * CPT, CHUNK)

    xp = jnp.concatenate(
        [in_feat, jnp.zeros((N_PAD - N_NODES, in_feat.shape[1]), f32)])

    half = W1.shape[1] // 2
    zeros64 = jnp.zeros((CHUNK, half), f32)

    deg = _make_deg_kernel()(src_p, dst_p)

    y1a, y1b, norms = pl.pallas_call(
        _tc_a_body,
        out_shape=(
            jax.ShapeDtypeStruct((N_PAD, half), f32),
            jax.ShapeDtypeStruct((N_PAD, half), f32),
            jax.ShapeDtypeStruct((2, N_PAD), f32),
        ),
    )(xp, W1, deg)

    agg = _make_agg_kernel(half)
    agg1a = agg(y1a, src_p, dst_p, zeros64)
    agg1b = agg(y1b, src_p, dst_p, zeros64)

    y2 = pl.pallas_call(
        _tc_b_body,
        out_shape=jax.ShapeDtypeStruct((N_PAD, W2.shape[1]), f32),
    )(agg1a, agg1b, norms, b1.reshape(1, -1), W2)

    agg2 = agg(y2, src_p, dst_p, zeros64)

    out = pl.pallas_call(
        _tc_c_body,
        out_shape=jax.ShapeDtypeStruct((N_NODES, W2.shape[1]), f32),
    )(agg2, norms, b2.reshape(1, -1))
    return out
